# Initial kernel scaffold; baseline (speedup 1.0000x reference)
import jax
import jax.numpy as jnp
from jax.experimental import pallas as pl


def kernel(tokens, emb_table, W1, b1, W2, b2):
    raise NotImplementedError("write your pallas kernel here")


# speedup vs baseline: 3.3969x; 3.3969x over previous
import functools

import jax
import jax.numpy as jnp
from jax import lax
from jax.experimental import pallas as pl
from jax.experimental.pallas import tpu as pltpu
from jax.experimental.pallas import tpu_sc as plsc

B = 16384
L = 200
D = 64
H = 128
C = 20

NUM_CORES = 2
NUM_SUBCORES = 16
NW = NUM_CORES * NUM_SUBCORES
BAGS_PER_W = B // NW
G = 2
GT = G * L
NG = BAGS_PER_W // G
NVREG = D // 16

_CHUNKS = []
_off = 0
while _off < GT:
    _sz = min(128, GT - _off)
    _CHUNKS.append((_off, _sz))
    _off += _sz


def _bag_body(tokens_hbm, table_hbm, out_hbm, idx_v, rows_v, out_v, sem_g, sem_t):
    wid = lax.axis_index("s") * NUM_CORES + lax.axis_index("c")
    tok_base = wid * BAGS_PER_W * L

    def tok_slice(g):
        return tokens_hbm.at[pl.ds(tok_base + g * GT, GT)]

    def fire_gathers(gslot, islot):
        for off, sz in _CHUNKS:
            pltpu.async_copy(
                table_hbm.at[idx_v.at[islot].at[pl.ds(off, sz)]],
                rows_v.at[gslot].at[pl.ds(off, sz)],
                sem_g,
            )

    def drain_gathers(gslot):
        for off, sz in _CHUNKS:
            pltpu.make_async_copy(
                table_hbm.at[pl.ds(0, sz)],
                rows_v.at[gslot].at[pl.ds(off, sz)],
                sem_g,
            ).wait()

    def drain_tokens(islot):
        pltpu.make_async_copy(
            tokens_hbm.at[pl.ds(0, GT)],
            idx_v.at[islot],
            sem_t,
        ).wait()

    pltpu.sync_copy(tok_slice(0), idx_v.at[0])
    fire_gathers(0, 0)
    pltpu.async_copy(tok_slice(1), idx_v.at[1], sem_t)

    def outer(i, carry):
        for j in range(4):
            g = i * 4 + j
            gslot, gslot_n = j % 2, (j + 1) % 2
            islot_n, islot_n2 = (j + 1) % 4, (j + 2) % 4

            @pl.when(g < NG - 1)
            def _():
                drain_tokens(islot_n)
                fire_gathers(gslot_n, islot_n)

            @pl.when(g < NG - 2)
            def _():
                pltpu.async_copy(tok_slice(g + 2), idx_v.at[islot_n2], sem_t)

            drain_gathers(gslot)

            for jj in range(G):
                def red_body(r, acc, _jj=jj, _gslot=gslot):
                    return tuple(
                        acc[c] + rows_v[_gslot, _jj * L + r, pl.ds(c * 16, 16)]
                        for c in range(NVREG)
                    )
                acc = lax.fori_loop(
                    0, L, red_body,
                    tuple(jnp.zeros((16,), jnp.float32) for _ in range(NVREG)),
                    unroll=8,
                )
                for c in range(NVREG):
                    out_v[g * G + jj, pl.ds(c * 16, 16)] = acc[c] * (1.0 / L)
        return carry

    lax.fori_loop(0, NG // 4, outer, 0)
    pltpu.sync_copy(out_v, out_hbm.at[pl.ds(wid * BAGS_PER_W, BAGS_PER_W)])


_bag_call = functools.partial(
    pl.kernel,
    out_type=jax.ShapeDtypeStruct((B, D), jnp.float32),
    mesh=plsc.VectorSubcoreMesh(core_axis_name="c", subcore_axis_name="s"),
    scratch_types=[
        pltpu.VMEM((4, GT), jnp.int32),
        pltpu.VMEM((2, GT, D), jnp.float32),
        pltpu.VMEM((BAGS_PER_W, D), jnp.float32),
        pltpu.SemaphoreType.DMA,
        pltpu.SemaphoreType.DMA,
    ],
    compiler_params=pltpu.CompilerParams(use_tc_tiling_on_sc=False),
)(_bag_body)


def _mlp_body(x_ref, w1_ref, b1_ref, w2_ref, b2_ref, o_ref):
    x = x_ref[...]
    h = jnp.dot(x, w1_ref[...], preferred_element_type=jnp.float32)
    h = jnp.maximum(h + b1_ref[...], 0.0)
    o_ref[...] = jnp.dot(h, w2_ref[...], preferred_element_type=jnp.float32) + b2_ref[...]


def _mlp_call(x, w1, b1, w2, b2):
    bt = 1024
    grid = (B // bt,)
    return pl.pallas_call(
        _mlp_body,
        grid=grid,
        in_specs=[
            pl.BlockSpec((bt, D), lambda i: (i, 0)),
            pl.BlockSpec((D, H), lambda i: (0, 0)),
            pl.BlockSpec((1, H), lambda i: (0, 0)),
            pl.BlockSpec((H, C), lambda i: (0, 0)),
            pl.BlockSpec((1, C), lambda i: (0, 0)),
        ],
        out_specs=pl.BlockSpec((bt, C), lambda i: (i, 0)),
        out_shape=jax.ShapeDtypeStruct((B, C), jnp.float32),
    )(x, w1, b1, w2, b2)


def kernel(tokens, emb_table, W1, b1, W2, b2):
    bags = _bag_call(tokens.reshape(-1), emb_table)
    return _mlp_call(bags, W1, b1.reshape(1, H), W2, b2.reshape(1, C))
